# Initial kernel scaffold; baseline (speedup 1.0000x reference)
import jax
import jax.numpy as jnp
from jax.experimental import pallas as pl


def kernel(x, support_indices, support_values, B_1, W):
    raise NotImplementedError("write your pallas kernel here")


# speedup vs baseline: 4.3132x; 4.3132x over previous
import functools

import jax
import jax.numpy as jnp
from jax import lax
from jax.experimental import pallas as pl
from jax.experimental.pallas import tpu as pltpu
from jax.experimental.pallas import tpu_sc as plsc

N = 10000
E = 320000
D = 128
NC = 2
NS = 16
NW = NC * NS
EPW = E // NW
CHUNK = 80
ROWS_PER_TILE = N // NS


def _sc_spmm(x, rows_idx, cols_idx, vals):
  mesh = plsc.VectorSubcoreMesh(
      core_axis_name="c", subcore_axis_name="s", num_cores=NC,
      num_subcores=NS)

  @functools.partial(
      pl.kernel,
      out_type=jax.ShapeDtypeStruct((NC, N, D), jnp.float32),
      mesh=mesh,
      scratch_types=[
          pltpu.VMEM((CHUNK,), jnp.int32),
          pltpu.VMEM((CHUNK,), jnp.int32),
          pltpu.VMEM((CHUNK,), jnp.float32),
          pltpu.VMEM((CHUNK, D), jnp.float32),
          pltpu.VMEM_SHARED((N, D), jnp.float32),
          pltpu.SemaphoreType.DMA,
      ],
  )
  def k(x_hbm, rows_hbm, cols_hbm, vals_hbm, out_hbm,
        cidx_v, ridx_v, vals_v, rows_v, acc_sh, sem):
    c = lax.axis_index("c")
    s = lax.axis_index("s")
    wid = s * NC + c

    nblk = N // CHUNK

    def _each_tile_block(fn):
      for k in range((nblk + NS - 1) // NS):
        b = s + k * NS

        @pl.when(b < nblk)
        def _(b=b):
          fn(b * CHUNK)

    zeros16 = jnp.zeros((16,), jnp.float32)

    @pl.loop(0, CHUNK)
    def _(e):
      for j in range(D // 16):
        rows_v[e, pl.ds(j * 16, 16)] = zeros16

    _each_tile_block(
        lambda r0: pltpu.sync_copy(rows_v, acc_sh.at[pl.ds(r0, CHUNK), :]))
    plsc.subcore_barrier()

    @pl.loop(0, EPW // CHUNK)
    def _(g):
      base = wid * EPW + g * CHUNK
      pltpu.sync_copy(cols_hbm.at[pl.ds(base, CHUNK)], cidx_v)
      pltpu.sync_copy(rows_hbm.at[pl.ds(base, CHUNK)], ridx_v)
      pltpu.sync_copy(vals_hbm.at[pl.ds(base, CHUNK)], vals_v)
      pltpu.async_copy(x_hbm.at[cidx_v], rows_v, sem).wait()

      @pl.loop(0, CHUNK // 16)
      def _(t):
        vvec = vals_v[pl.ds(t * 16, 16)]
        for l in range(16):
          v = vvec[l]
          e = t * 16 + l
          for j in range(D // 16):
            sl = pl.ds(j * 16, 16)
            rows_v[e, sl] = rows_v[e, sl] * v

      pltpu.sync_copy(rows_v, acc_sh.at[ridx_v], add=True)

    plsc.subcore_barrier()

    _each_tile_block(
        lambda r0: pltpu.sync_copy(acc_sh.at[pl.ds(r0, CHUNK), :],
                                   out_hbm.at[c, pl.ds(r0, CHUNK), :]))

  return k(x, rows_idx, cols_idx, vals)


MB = 256


def _tc_body(b1_ref, p_ref, w_ref, relu_ref, z1_ref):
  psum = p_ref[0] + p_ref[1]
  t = jnp.dot(b1_ref[...], psum, preferred_element_type=jnp.float32)
  z1 = jnp.dot(t, w_ref[...], preferred_element_type=jnp.float32)
  z1_ref[...] = z1
  relu_ref[...] = jnp.maximum(z1, 0.0)


def _tc_matmuls(B_1, partials, W):
  nb = B_1.shape[0]
  grid = nb // MB
  return pl.pallas_call(
      _tc_body,
      grid=(grid,),
      in_specs=[
          pl.BlockSpec((MB, N), lambda i: (i, 0)),
          pl.BlockSpec((NC, N, D), lambda i: (0, 0, 0)),
          pl.BlockSpec((D, D), lambda i: (0, 0)),
      ],
      out_specs=[
          pl.BlockSpec((MB, D), lambda i: (i, 0)),
          pl.BlockSpec((MB, D), lambda i: (i, 0)),
      ],
      out_shape=[
          jax.ShapeDtypeStruct((nb, D), jnp.float32),
          jax.ShapeDtypeStruct((nb, D), jnp.float32),
      ],
      compiler_params=pltpu.CompilerParams(
          dimension_semantics=("arbitrary",)),
  )(B_1, partials, W)


def kernel(x, support_indices, support_values, B_1, W):
  rows_idx = support_indices[0]
  cols_idx = support_indices[1]
  partials = _sc_spmm(x, rows_idx, cols_idx, support_values)
  relu_out, z1 = _tc_matmuls(B_1, partials, W)
  return (relu_out, z1)


# speedup vs baseline: 8.0910x; 1.8759x over previous
import functools

import jax
import jax.numpy as jnp
from jax import lax
from jax.experimental import pallas as pl
from jax.experimental.pallas import tpu as pltpu
from jax.experimental.pallas import tpu_sc as plsc

N = 10000
E = 320000
D = 128
NC = 2
NS = 16
NW = NC * NS
EPW = E // NW
CHUNK = 80
GC = EPW // CHUNK


def _sc_spmm(x, rows_idx, cols_idx, vals):
  mesh = plsc.VectorSubcoreMesh(
      core_axis_name="c", subcore_axis_name="s", num_cores=NC,
      num_subcores=NS)

  @functools.partial(
      pl.kernel,
      out_type=jax.ShapeDtypeStruct((NC, N, D), jnp.float32),
      mesh=mesh,
      scratch_types=[
          pltpu.VMEM((EPW,), jnp.int32),
          pltpu.VMEM((EPW,), jnp.int32),
          pltpu.VMEM((EPW,), jnp.float32),
          pltpu.VMEM((CHUNK, D), jnp.float32),
          pltpu.VMEM((CHUNK, D), jnp.float32),
          pltpu.VMEM_SHARED((N, D), jnp.float32),
          pltpu.SemaphoreType.DMA,
          pltpu.SemaphoreType.DMA,
      ],
  )
  def k(x_hbm, rows_hbm, cols_hbm, vals_hbm, out_hbm,
        cidx_v, ridx_v, vals_v, buf0, buf1, acc_sh, gsem, ssem):
    c = lax.axis_index("c")
    s = lax.axis_index("s")
    wid = s * NC + c
    bufs = (buf0, buf1)

    nblk = N // CHUNK

    def _each_tile_block(fn):
      for kk in range((nblk + NS - 1) // NS):
        b = s + kk * NS

        @pl.when(b < nblk)
        def _(b=b):
          fn(b * CHUNK)

    zeros16 = jnp.zeros((16,), jnp.float32)

    @pl.loop(0, CHUNK)
    def _(e):
      for j in range(D // 16):
        buf0[e, pl.ds(j * 16, 16)] = zeros16

    _each_tile_block(
        lambda r0: pltpu.sync_copy(buf0, acc_sh.at[pl.ds(r0, CHUNK), :]))

    base = wid * EPW
    pltpu.sync_copy(cols_hbm.at[pl.ds(base, EPW)], cidx_v)
    pltpu.sync_copy(rows_hbm.at[pl.ds(base, EPW)], ridx_v)
    pltpu.sync_copy(vals_hbm.at[pl.ds(base, EPW)], vals_v)
    plsc.subcore_barrier()

    def _gather_start(g, buf):
      pltpu.async_copy(x_hbm.at[cidx_v.at[pl.ds(g * CHUNK, CHUNK)]],
                       buf, gsem)

    def _gather_wait(buf):
      pltpu.make_async_copy(x_hbm.at[cidx_v.at[pl.ds(0, CHUNK)]],
                            buf, gsem).wait()

    def _scatter_start(g, buf):
      for t in range(CHUNK // 16):
        idx = ridx_v[pl.ds(g * CHUNK + t * 16, 16)]
        pltpu.async_copy(buf.at[pl.ds(t * 16, 16), :],
                         acc_sh.at[idx], ssem, add=True)

    def _scatter_wait(buf):
      for t in range(CHUNK // 16):
        idx = ridx_v[pl.ds(t * 16, 16)]
        pltpu.make_async_copy(buf.at[pl.ds(t * 16, 16), :],
                              acc_sh.at[idx], ssem).wait()

    def _scale(g, buf):
      for t in range(CHUNK // 16):
        vv = vals_v[pl.ds(g * CHUNK + t * 16, 16)]
        for l in range(16):
          e = t * 16 + l
          v = vv[l]
          for j in range(D // 16):
            sl = pl.ds(j * 16, 16)
            buf[e, sl] = buf[e, sl] * v

    _gather_start(0, buf0)
    _gather_wait(buf0)
    _gather_start(1, buf1)
    _scale(0, buf0)
    _scatter_start(0, buf0)

    @pl.loop(0, (GC - 1) // 2)
    def _(i):
      for h in range(2):
        g = 1 + 2 * i + h
        act = bufs[(1 + h) % 2]
        oth = bufs[h % 2]
        _gather_wait(act)
        _scatter_wait(oth)

        @pl.when(g + 1 < GC)
        def _(g=g, oth=oth):
          _gather_start(g + 1, oth)

        _scale(g, act)
        _scatter_start(g, act)

    _scatter_wait(bufs[(GC - 1) % 2])
    plsc.subcore_barrier()

    _each_tile_block(
        lambda r0: pltpu.sync_copy(acc_sh.at[pl.ds(r0, CHUNK), :],
                                   out_hbm.at[c, pl.ds(r0, CHUNK), :]))

  return k(x, rows_idx, cols_idx, vals)


MB = 256


def _tc_body(b1_ref, p_ref, w_ref, relu_ref, z1_ref):
  psum = p_ref[0] + p_ref[1]
  t = jnp.dot(b1_ref[...], psum, preferred_element_type=jnp.float32)
  z1 = jnp.dot(t, w_ref[...], preferred_element_type=jnp.float32)
  z1_ref[...] = z1
  relu_ref[...] = jnp.maximum(z1, 0.0)


def _tc_matmuls(B_1, partials, W):
  nb = B_1.shape[0]
  grid = nb // MB
  return pl.pallas_call(
      _tc_body,
      grid=(grid,),
      in_specs=[
          pl.BlockSpec((MB, N), lambda i: (i, 0)),
          pl.BlockSpec((NC, N, D), lambda i: (0, 0, 0)),
          pl.BlockSpec((D, D), lambda i: (0, 0)),
      ],
      out_specs=[
          pl.BlockSpec((MB, D), lambda i: (i, 0)),
          pl.BlockSpec((MB, D), lambda i: (i, 0)),
      ],
      out_shape=[
          jax.ShapeDtypeStruct((nb, D), jnp.float32),
          jax.ShapeDtypeStruct((nb, D), jnp.float32),
      ],
      compiler_params=pltpu.CompilerParams(
          dimension_semantics=("arbitrary",)),
  )(B_1, partials, W)


def kernel(x, support_indices, support_values, B_1, W):
  rows_idx = support_indices[0]
  cols_idx = support_indices[1]
  partials = _sc_spmm(x, rows_idx, cols_idx, support_values)
  relu_out, z1 = _tc_matmuls(B_1, partials, W)
  return (relu_out, z1)
